# Initial kernel scaffold; baseline (speedup 1.0000x reference)
import functools

import jax
import jax.numpy as jnp
from jax import lax
from jax.experimental import pallas as pl
from jax.experimental.pallas import tpu as pltpu
from jax.experimental.pallas import tpu_sc as plsc

N = 10000
D = 128
E = 320000
C_OUT = 64
NC = 2
NS = 16
NW = NC * NS
EPW = E // NW
CH = 80
NCHUNK = EPW // CH
RPT = N // NS
CL = 16

_mesh = plsc.VectorSubcoreMesh(core_axis_name="c", subcore_axis_name="s")


def _sc_agg_count_body(table, src_h, dst_h, zfeat_h, zcnt_h, acc_out, cnt_out,
                       src_v, dst_v, rows_v, ones_v, acc_sh, cnt_sh, sem):
    c = lax.axis_index("c")
    s = lax.axis_index("s")
    wid = s * NC + c
    rbase = s * RPT
    pltpu.sync_copy(zfeat_h.at[pl.ds(rbase, RPT)], acc_sh.at[pl.ds(rbase, RPT)])
    pltpu.sync_copy(zcnt_h.at[pl.ds(rbase, RPT)], cnt_sh.at[pl.ds(rbase, RPT)])

    @pl.loop(0, CH)
    def _(i):
        ones_v[i] = jnp.full((CL,), 1.0, jnp.float32)

    plsc.subcore_barrier()

    ebase = wid * EPW

    @pl.loop(0, NCHUNK)
    def _(ci):
        o = ebase + ci * CH
        pltpu.sync_copy(src_h.at[pl.ds(o, CH)], src_v)
        pltpu.sync_copy(dst_h.at[pl.ds(o, CH)], dst_v)
        pltpu.async_copy(table.at[src_v], rows_v, sem).wait()
        pltpu.sync_copy(rows_v, acc_sh.at[dst_v], add=True)
        pltpu.sync_copy(ones_v, cnt_sh.at[dst_v], add=True)

    plsc.subcore_barrier()
    pltpu.sync_copy(acc_sh.at[pl.ds(rbase, RPT)],
                    acc_out.at[c, pl.ds(rbase, RPT)])
    pltpu.sync_copy(cnt_sh.at[pl.ds(rbase, RPT)],
                    cnt_out.at[c, pl.ds(rbase, RPT)])


def _sc_agg_body(table, src_h, dst_h, zfeat_h, acc_out,
                 src_v, dst_v, rows_v, acc_sh, sem):
    c = lax.axis_index("c")
    s = lax.axis_index("s")
    wid = s * NC + c
    rbase = s * RPT
    pltpu.sync_copy(zfeat_h.at[pl.ds(rbase, RPT)], acc_sh.at[pl.ds(rbase, RPT)])
    plsc.subcore_barrier()

    ebase = wid * EPW

    @pl.loop(0, NCHUNK)
    def _(ci):
        o = ebase + ci * CH
        pltpu.sync_copy(src_h.at[pl.ds(o, CH)], src_v)
        pltpu.sync_copy(dst_h.at[pl.ds(o, CH)], dst_v)
        pltpu.async_copy(table.at[src_v], rows_v, sem).wait()
        pltpu.sync_copy(rows_v, acc_sh.at[dst_v], add=True)

    plsc.subcore_barrier()
    pltpu.sync_copy(acc_sh.at[pl.ds(rbase, RPT)],
                    acc_out.at[c, pl.ds(rbase, RPT)])


_sc_agg_count = pl.kernel(
    _sc_agg_count_body,
    out_type=(jax.ShapeDtypeStruct((NC, N, D), jnp.float32),
              jax.ShapeDtypeStruct((NC, N, CL), jnp.float32)),
    mesh=_mesh,
    scratch_types=[
        pltpu.VMEM((CH,), jnp.int32),
        pltpu.VMEM((CH,), jnp.int32),
        pltpu.VMEM((CH, D), jnp.float32),
        pltpu.VMEM((CH, CL), jnp.float32),
        pltpu.VMEM_SHARED((N, D), jnp.float32),
        pltpu.VMEM_SHARED((N, CL), jnp.float32),
        pltpu.SemaphoreType.DMA,
    ],
)

_sc_agg = pl.kernel(
    _sc_agg_body,
    out_type=jax.ShapeDtypeStruct((NC, N, D), jnp.float32),
    mesh=_mesh,
    scratch_types=[
        pltpu.VMEM((CH,), jnp.int32),
        pltpu.VMEM((CH,), jnp.int32),
        pltpu.VMEM((CH, D), jnp.float32),
        pltpu.VMEM_SHARED((N, D), jnp.float32),
        pltpu.SemaphoreType.DMA,
    ],
)


_DN = (((1,), (1,)), ((), ()))


def _dense1_kernel(acc, cnt, x, w1l, b1l, w1r, out):
    a = acc[...]
    ssum = a[0] + a[1]
    cc = cnt[...]
    deg = cc[0, :, :1] + cc[1, :, :1]
    mean = ssum / jnp.maximum(deg, 1.0)
    h = lax.dot_general(mean, w1l[...], _DN, preferred_element_type=jnp.float32)
    h = h + lax.dot_general(x[...], w1r[...], _DN,
                            preferred_element_type=jnp.float32)
    h = h + b1l[...]
    out[...] = jnp.maximum(h, 0.0)


def _dense2_kernel(acc, cnt, h1, w2l, b2l, w2r, wlin, blin, out):
    a = acc[...]
    ssum = a[0] + a[1]
    cc = cnt[...]
    deg = cc[0, :, :1] + cc[1, :, :1]
    mean = ssum / jnp.maximum(deg, 1.0)
    h = lax.dot_general(mean, w2l[...], _DN, preferred_element_type=jnp.float32)
    h = h + lax.dot_general(h1[...], w2r[...], _DN,
                            preferred_element_type=jnp.float32)
    h = jnp.maximum(h + b2l[...], 0.0)
    out[...] = lax.dot_general(h, wlin[...], _DN,
                               preferred_element_type=jnp.float32) + blin[...]


_R = 1000

_dense1 = pl.pallas_call(
    _dense1_kernel,
    grid=(N // _R,),
    in_specs=[
        pl.BlockSpec((NC, _R, D), lambda i: (0, i, 0)),
        pl.BlockSpec((NC, _R, CL), lambda i: (0, i, 0)),
        pl.BlockSpec((_R, D), lambda i: (i, 0)),
        pl.BlockSpec((D, D), lambda i: (0, 0)),
        pl.BlockSpec((1, D), lambda i: (0, 0)),
        pl.BlockSpec((D, D), lambda i: (0, 0)),
    ],
    out_specs=pl.BlockSpec((_R, D), lambda i: (i, 0)),
    out_shape=jax.ShapeDtypeStruct((N, D), jnp.float32),
)

_dense2 = pl.pallas_call(
    _dense2_kernel,
    grid=(N // _R,),
    in_specs=[
        pl.BlockSpec((NC, _R, D), lambda i: (0, i, 0)),
        pl.BlockSpec((NC, _R, CL), lambda i: (0, i, 0)),
        pl.BlockSpec((_R, D), lambda i: (i, 0)),
        pl.BlockSpec((D, D), lambda i: (0, 0)),
        pl.BlockSpec((1, D), lambda i: (0, 0)),
        pl.BlockSpec((D, D), lambda i: (0, 0)),
        pl.BlockSpec((C_OUT, D), lambda i: (0, 0)),
        pl.BlockSpec((1, C_OUT), lambda i: (0, 0)),
    ],
    out_specs=pl.BlockSpec((_R, C_OUT), lambda i: (i, 0)),
    out_shape=jax.ShapeDtypeStruct((N, C_OUT), jnp.float32),
)


def kernel(x, edge_index, W1l, b1l, W1r, W2l, b2l, W2r, Wlin, blin):
    src = edge_index[0]
    dst = edge_index[1]
    zfeat = jnp.zeros((N, D), jnp.float32)
    zcnt = jnp.zeros((N, CL), jnp.float32)
    acc1, cnt = _sc_agg_count(x, src, dst, zfeat, zcnt)
    h1 = _dense1(acc1, cnt, x, W1l, b1l.reshape(1, D), W1r)
    acc2 = _sc_agg(h1, src, dst, zfeat)
    return _dense2(acc2, cnt, h1, W2l, b2l.reshape(1, D), W2r,
                   Wlin, blin.reshape(1, C_OUT))


# speedup vs baseline: 5.5411x; 5.5411x over previous
import dataclasses
import functools

import jax
import jax.numpy as jnp
from jax import lax
from jax.experimental import pallas as pl
from jax.experimental.pallas import tpu as pltpu
from jax.experimental.pallas import tpu_sc as plsc

N = 10000
D = 128
E = 320000
C_OUT = 64
NC = 2
NS = 16
NW = NC * NS
EPW = E // NW
CH = 80
NCHUNK = EPW // CH
RPT = 624
TAIL = N - NS * RPT
CL = 16

_mesh = plsc.VectorSubcoreMesh(core_axis_name="c", subcore_axis_name="s")


def _copy_rows(s, src_at, dst_at):
    rbase = s * RPT
    pltpu.sync_copy(src_at(pl.ds(rbase, RPT)), dst_at(pl.ds(rbase, RPT)))

    @pl.when(s == 0)
    def _():
        pltpu.sync_copy(src_at(pl.ds(NS * RPT, TAIL)),
                        dst_at(pl.ds(NS * RPT, TAIL)))


def _sc_agg_count_body(table, src_h, dst_h, zfeat_h, zcnt_h, acc_out, cnt_out,
                       src_v, dst_v, rows_v, cnt_local, acc_sh, sem):
    c = lax.axis_index("c")
    s = lax.axis_index("s")
    wid = s * NC + c
    _copy_rows(s, lambda d: zfeat_h.at[d], lambda d: acc_sh.at[d])
    pltpu.sync_copy(zcnt_h, cnt_local)
    plsc.subcore_barrier()

    ebase = wid * EPW
    ones = jnp.full((16,), 1.0, jnp.float32)

    @pl.loop(0, NCHUNK)
    def _(ci):
        o = ebase + ci * CH
        pltpu.sync_copy(src_h.at[pl.ds(o, CH)], src_v)
        pltpu.sync_copy(dst_h.at[pl.ds(o, CH)], dst_v)
        pltpu.async_copy(table.at[src_v], rows_v, sem).wait()
        pltpu.sync_copy(rows_v, acc_sh.at[dst_v], add=True)
        for g in range(CH // 16):
            idx = dst_v[pl.ds(g * 16, 16)]
            plsc.addupdate_scatter(cnt_local, [idx], ones)

    plsc.subcore_barrier()
    _copy_rows(s, lambda d: acc_sh.at[d], lambda d: acc_out.at[c, d])
    pltpu.sync_copy(cnt_local, cnt_out.at[pl.ds(wid * N, N)])


def _sc_agg_body(table, src_h, dst_h, zfeat_h, acc_out,
                 src_v, dst_v, rows_v, acc_sh, sem):
    c = lax.axis_index("c")
    s = lax.axis_index("s")
    wid = s * NC + c
    _copy_rows(s, lambda d: zfeat_h.at[d], lambda d: acc_sh.at[d])
    plsc.subcore_barrier()

    ebase = wid * EPW

    @pl.loop(0, NCHUNK)
    def _(ci):
        o = ebase + ci * CH
        pltpu.sync_copy(src_h.at[pl.ds(o, CH)], src_v)
        pltpu.sync_copy(dst_h.at[pl.ds(o, CH)], dst_v)
        pltpu.async_copy(table.at[src_v], rows_v, sem).wait()
        pltpu.sync_copy(rows_v, acc_sh.at[dst_v], add=True)

    plsc.subcore_barrier()
    _copy_rows(s, lambda d: acc_sh.at[d], lambda d: acc_out.at[c, d])


_sc_agg_count = pl.kernel(
    _sc_agg_count_body,
    out_type=(jax.ShapeDtypeStruct((NC, N, D), jnp.float32),
              jax.ShapeDtypeStruct((NW * N,), jnp.float32)),
    mesh=_mesh,
    scratch_types=[
        pltpu.VMEM((CH,), jnp.int32),
        pltpu.VMEM((CH,), jnp.int32),
        pltpu.VMEM((CH, D), jnp.float32),
        pltpu.VMEM((N,), jnp.float32),
        pltpu.VMEM_SHARED((N, D), jnp.float32),
        pltpu.SemaphoreType.DMA,
    ],
    compiler_params=dataclasses.replace(pltpu.CompilerParams(),
                                        needs_layout_passes=False)
    if "needs_layout_passes" in pltpu.CompilerParams.__dataclass_fields__
    else None,
)

_sc_agg = pl.kernel(
    _sc_agg_body,
    out_type=jax.ShapeDtypeStruct((NC, N, D), jnp.float32),
    mesh=_mesh,
    scratch_types=[
        pltpu.VMEM((CH,), jnp.int32),
        pltpu.VMEM((CH,), jnp.int32),
        pltpu.VMEM((CH, D), jnp.float32),
        pltpu.VMEM_SHARED((N, D), jnp.float32),
        pltpu.SemaphoreType.DMA,
    ],
)


_DN = (((1,), (1,)), ((), ()))


def _dense1_kernel(acc, cnt, x, w1l, b1l, w1r, out):
    a = acc[...]
    ssum = a[0] + a[1]
    deg = jnp.sum(cnt[...].reshape(NW, -1), axis=0)[:, None]
    mean = ssum / jnp.maximum(deg, 1.0)
    h = lax.dot_general(mean, w1l[...], _DN, preferred_element_type=jnp.float32)
    h = h + lax.dot_general(x[...], w1r[...], _DN,
                            preferred_element_type=jnp.float32)
    h = h + b1l[...]
    out[...] = jnp.maximum(h, 0.0)


def _dense2_kernel(acc, cnt, h1, w2l, b2l, w2r, wlin, blin, out):
    a = acc[...]
    ssum = a[0] + a[1]
    deg = jnp.sum(cnt[...].reshape(NW, -1), axis=0)[:, None]
    mean = ssum / jnp.maximum(deg, 1.0)
    h = lax.dot_general(mean, w2l[...], _DN, preferred_element_type=jnp.float32)
    h = h + lax.dot_general(h1[...], w2r[...], _DN,
                            preferred_element_type=jnp.float32)
    h = jnp.maximum(h + b2l[...], 0.0)
    out[...] = lax.dot_general(h, wlin[...], _DN,
                               preferred_element_type=jnp.float32) + blin[...]


_R = 1000

_dense1 = pl.pallas_call(
    _dense1_kernel,
    grid=(N // _R,),
    in_specs=[
        pl.BlockSpec((NC, _R, D), lambda i: (0, i, 0)),
        pl.BlockSpec((NW, 1, 1, _R), lambda i: (0, i, 0, 0)),
        pl.BlockSpec((_R, D), lambda i: (i, 0)),
        pl.BlockSpec((D, D), lambda i: (0, 0)),
        pl.BlockSpec((1, D), lambda i: (0, 0)),
        pl.BlockSpec((D, D), lambda i: (0, 0)),
    ],
    out_specs=pl.BlockSpec((_R, D), lambda i: (i, 0)),
    out_shape=jax.ShapeDtypeStruct((N, D), jnp.float32),
)

_dense2 = pl.pallas_call(
    _dense2_kernel,
    grid=(N // _R,),
    in_specs=[
        pl.BlockSpec((NC, _R, D), lambda i: (0, i, 0)),
        pl.BlockSpec((NW, 1, 1, _R), lambda i: (0, i, 0, 0)),
        pl.BlockSpec((_R, D), lambda i: (i, 0)),
        pl.BlockSpec((D, D), lambda i: (0, 0)),
        pl.BlockSpec((1, D), lambda i: (0, 0)),
        pl.BlockSpec((D, D), lambda i: (0, 0)),
        pl.BlockSpec((C_OUT, D), lambda i: (0, 0)),
        pl.BlockSpec((1, C_OUT), lambda i: (0, 0)),
    ],
    out_specs=pl.BlockSpec((_R, C_OUT), lambda i: (i, 0)),
    out_shape=jax.ShapeDtypeStruct((N, C_OUT), jnp.float32),
)


def kernel(x, edge_index, W1l, b1l, W1r, W2l, b2l, W2r, Wlin, blin):
    src = edge_index[0]
    dst = edge_index[1]
    zfeat = jnp.zeros((N, D), jnp.float32)
    zcnt = jnp.zeros((N,), jnp.float32)
    acc1, cnt = _sc_agg_count(x, src, dst, zfeat, zcnt)
    cnt = cnt.reshape(NW, N // _R, 1, _R)
    h1 = _dense1(acc1, cnt, x, W1l, b1l.reshape(1, D), W1r)
    acc2 = _sc_agg(h1, src, dst, zfeat)
    return _dense2(acc2, cnt, h1, W2l, b2l.reshape(1, D), W2r,
                   Wlin, blin.reshape(1, C_OUT))


# speedup vs baseline: 12.5006x; 2.2560x over previous
import dataclasses
import functools

import jax
import jax.numpy as jnp
from jax import lax
from jax.experimental import pallas as pl
from jax.experimental.pallas import tpu as pltpu
from jax.experimental.pallas import tpu_sc as plsc

N = 10000
D = 128
E = 320000
C_OUT = 64
NC = 2
NS = 16
NW = NC * NS
EPW = E // NW
CH = 80
NCHUNK = EPW // CH
RPT = 624
TAIL = N - NS * RPT
CL = 16

_mesh = plsc.VectorSubcoreMesh(core_axis_name="c", subcore_axis_name="s")


def _copy_rows(s, src_at, dst_at):
    rbase = s * RPT
    pltpu.sync_copy(src_at(pl.ds(rbase, RPT)), dst_at(pl.ds(rbase, RPT)))

    @pl.when(s == 0)
    def _():
        pltpu.sync_copy(src_at(pl.ds(NS * RPT, TAIL)),
                        dst_at(pl.ds(NS * RPT, TAIL)))


NBUF = 3
NMAIN = (NCHUNK // NBUF) * NBUF


def _agg_pipeline(table, src_h, dst_h, ebase, acc_sh,
                  srcv, dstv, isems, rows, gsems, count_fn):
    def start_idx(k, b):
        o = ebase + k * CH
        pltpu.make_async_copy(src_h.at[pl.ds(o, CH)], srcv[b], isems[b]).start()
        pltpu.make_async_copy(dst_h.at[pl.ds(o, CH)], dstv[b], isems[b]).start()

    def wait_idx(b):
        pltpu.make_async_copy(src_h.at[pl.ds(0, CH)], srcv[b], isems[b]).wait()
        pltpu.make_async_copy(dst_h.at[pl.ds(0, CH)], dstv[b], isems[b]).wait()

    def start_gather(b):
        wait_idx(b)
        pltpu.make_async_copy(table.at[srcv[b]], rows[b], gsems[b]).start()

    def drain(k, b):
        pltpu.make_async_copy(table.at[pl.ds(0, CH)], rows[b], gsems[b]).wait()
        pltpu.sync_copy(rows[b], acc_sh.at[dstv[b]], add=True)
        count_fn(k, dstv[b])

    for b in range(NBUF):
        start_idx(b, b)
    for b in range(NBUF - 1):
        start_gather(b)

    @pl.loop(0, NMAIN, step=NBUF)
    def _(ci):
        for b in range(NBUF):
            k = ci + b
            drain(k, b)
            nk = k + NBUF

            @pl.when(nk < NCHUNK)
            def _():
                start_idx(nk, b)

            @pl.when(k + NBUF - 1 < NCHUNK)
            def _():
                start_gather((b + NBUF - 1) % NBUF)

    for t in range(NMAIN, NCHUNK):
        drain(t, t % NBUF)


def _sc_agg_count_body(table, src_h, dst_h, zfeat_h, zcnt_h, acc_out, cnt_out,
                       cnt_local, acc_sh,
                       s0, s1, s2, d0, d1, d2, i0, i1, i2,
                       r0, r1, r2, g0, g1, g2):
    c = lax.axis_index("c")
    s = lax.axis_index("s")
    wid = s * NC + c
    _copy_rows(s, lambda d: zfeat_h.at[d], lambda d: acc_sh.at[d])
    pltpu.sync_copy(zcnt_h, cnt_local)
    plsc.subcore_barrier()

    ones = jnp.full((16,), 1.0, jnp.float32)

    def count_fn(k, dref):
        for g in range(CH // 16):
            plsc.addupdate_scatter(cnt_local, [dref[pl.ds(g * 16, 16)]], ones)

    _agg_pipeline(table, src_h, dst_h, wid * EPW, acc_sh,
                  [s0, s1, s2], [d0, d1, d2], [i0, i1, i2],
                  [r0, r1, r2], [g0, g1, g2], count_fn)

    plsc.subcore_barrier()
    _copy_rows(s, lambda d: acc_sh.at[d], lambda d: acc_out.at[c, d])
    pltpu.sync_copy(cnt_local, cnt_out.at[pl.ds(wid * N, N)])


def _sc_agg_body(table, src_h, dst_h, zfeat_h, acc_out,
                 acc_sh,
                 s0, s1, s2, d0, d1, d2, i0, i1, i2,
                 r0, r1, r2, g0, g1, g2):
    c = lax.axis_index("c")
    s = lax.axis_index("s")
    wid = s * NC + c
    _copy_rows(s, lambda d: zfeat_h.at[d], lambda d: acc_sh.at[d])
    plsc.subcore_barrier()

    _agg_pipeline(table, src_h, dst_h, wid * EPW, acc_sh,
                  [s0, s1, s2], [d0, d1, d2], [i0, i1, i2],
                  [r0, r1, r2], [g0, g1, g2], lambda k, dref: None)

    plsc.subcore_barrier()
    _copy_rows(s, lambda d: acc_sh.at[d], lambda d: acc_out.at[c, d])


_idx_ring = ([pltpu.VMEM((CH,), jnp.int32)] * (2 * NBUF)
             + [pltpu.SemaphoreType.DMA] * NBUF)
_row_ring = ([pltpu.VMEM((CH, D), jnp.float32)] * NBUF
             + [pltpu.SemaphoreType.DMA] * NBUF)

_sc_agg_count = pl.kernel(
    _sc_agg_count_body,
    out_type=(jax.ShapeDtypeStruct((NC, N, D), jnp.float32),
              jax.ShapeDtypeStruct((NW * N,), jnp.float32)),
    mesh=_mesh,
    scratch_types=(
        [pltpu.VMEM((N,), jnp.float32),
         pltpu.VMEM_SHARED((N, D), jnp.float32)]
        + _idx_ring + _row_ring
    ),
    compiler_params=dataclasses.replace(pltpu.CompilerParams(),
                                        needs_layout_passes=False)
    if "needs_layout_passes" in pltpu.CompilerParams.__dataclass_fields__
    else None,
)

_sc_agg = pl.kernel(
    _sc_agg_body,
    out_type=jax.ShapeDtypeStruct((NC, N, D), jnp.float32),
    mesh=_mesh,
    scratch_types=(
        [pltpu.VMEM_SHARED((N, D), jnp.float32)]
        + _idx_ring + _row_ring
    ),
)


_DN = (((1,), (1,)), ((), ()))


def _dense1_kernel(acc, cnt, x, w1l, b1l, w1r, out):
    a = acc[...]
    ssum = a[0] + a[1]
    deg = jnp.sum(cnt[...].reshape(NW, -1), axis=0)[:, None]
    mean = ssum / jnp.maximum(deg, 1.0)
    h = lax.dot_general(mean, w1l[...], _DN, preferred_element_type=jnp.float32)
    h = h + lax.dot_general(x[...], w1r[...], _DN,
                            preferred_element_type=jnp.float32)
    h = h + b1l[...]
    out[...] = jnp.maximum(h, 0.0)


def _dense2_kernel(acc, cnt, h1, w2l, b2l, w2r, wlin, blin, out):
    a = acc[...]
    ssum = a[0] + a[1]
    deg = jnp.sum(cnt[...].reshape(NW, -1), axis=0)[:, None]
    mean = ssum / jnp.maximum(deg, 1.0)
    h = lax.dot_general(mean, w2l[...], _DN, preferred_element_type=jnp.float32)
    h = h + lax.dot_general(h1[...], w2r[...], _DN,
                            preferred_element_type=jnp.float32)
    h = jnp.maximum(h + b2l[...], 0.0)
    out[...] = lax.dot_general(h, wlin[...], _DN,
                               preferred_element_type=jnp.float32) + blin[...]


_R = 1000

_dense1 = pl.pallas_call(
    _dense1_kernel,
    grid=(N // _R,),
    in_specs=[
        pl.BlockSpec((NC, _R, D), lambda i: (0, i, 0)),
        pl.BlockSpec((NW, 1, 1, _R), lambda i: (0, i, 0, 0)),
        pl.BlockSpec((_R, D), lambda i: (i, 0)),
        pl.BlockSpec((D, D), lambda i: (0, 0)),
        pl.BlockSpec((1, D), lambda i: (0, 0)),
        pl.BlockSpec((D, D), lambda i: (0, 0)),
    ],
    out_specs=pl.BlockSpec((_R, D), lambda i: (i, 0)),
    out_shape=jax.ShapeDtypeStruct((N, D), jnp.float32),
)

_dense2 = pl.pallas_call(
    _dense2_kernel,
    grid=(N // _R,),
    in_specs=[
        pl.BlockSpec((NC, _R, D), lambda i: (0, i, 0)),
        pl.BlockSpec((NW, 1, 1, _R), lambda i: (0, i, 0, 0)),
        pl.BlockSpec((_R, D), lambda i: (i, 0)),
        pl.BlockSpec((D, D), lambda i: (0, 0)),
        pl.BlockSpec((1, D), lambda i: (0, 0)),
        pl.BlockSpec((D, D), lambda i: (0, 0)),
        pl.BlockSpec((C_OUT, D), lambda i: (0, 0)),
        pl.BlockSpec((1, C_OUT), lambda i: (0, 0)),
    ],
    out_specs=pl.BlockSpec((_R, C_OUT), lambda i: (i, 0)),
    out_shape=jax.ShapeDtypeStruct((N, C_OUT), jnp.float32),
)


def kernel(x, edge_index, W1l, b1l, W1r, W2l, b2l, W2r, Wlin, blin):
    src = edge_index[0]
    dst = edge_index[1]
    zfeat = jnp.zeros((N, D), jnp.float32)
    zcnt = jnp.zeros((N,), jnp.float32)
    acc1, cnt = _sc_agg_count(x, src, dst, zfeat, zcnt)
    cnt = cnt.reshape(NW, N // _R, 1, _R)
    h1 = _dense1(acc1, cnt, x, W1l, b1l.reshape(1, D), W1r)
    acc2 = _sc_agg(h1, src, dst, zfeat)
    return _dense2(acc2, cnt, h1, W2l, b2l.reshape(1, D), W2r,
                   Wlin, blin.reshape(1, C_OUT))


# speedup vs baseline: 14.0755x; 1.1260x over previous
import dataclasses
import functools

import jax
import jax.numpy as jnp
from jax import lax
from jax.experimental import pallas as pl
from jax.experimental.pallas import tpu as pltpu
from jax.experimental.pallas import tpu_sc as plsc

N = 10000
D = 128
E = 320000
C_OUT = 64
NC = 2
NS = 16
NW = NC * NS
EPW = E // NW
CH = 80
NCHUNK = EPW // CH
RPT = 624
TAIL = N - NS * RPT
CL = 16

_mesh = plsc.VectorSubcoreMesh(core_axis_name="c", subcore_axis_name="s")


def _copy_rows(s, src_at, dst_at):
    rbase = s * RPT
    pltpu.sync_copy(src_at(pl.ds(rbase, RPT)), dst_at(pl.ds(rbase, RPT)))

    @pl.when(s == 0)
    def _():
        pltpu.sync_copy(src_at(pl.ds(NS * RPT, TAIL)),
                        dst_at(pl.ds(NS * RPT, TAIL)))


NBUF = 3
NTRASH = N


def _make_sc_agg(ch, with_count):
    nchunk = -(-EPW // ch)
    ept = nchunk * ch
    nmain = (nchunk // NBUF) * NBUF

    def body(table, src_h, dst_h, zfeat_h, *rest):
        if with_count:
            (zcnt_h, acc_out, cnt_out, cnt_local, acc_sh, *rings) = rest
        else:
            (acc_out, acc_sh, *rings) = rest
        srcv = rings[0:NBUF]
        dstv = rings[NBUF:2 * NBUF]
        dsc = rings[2 * NBUF:3 * NBUF]
        isems = rings[3 * NBUF:4 * NBUF]
        rows = rings[4 * NBUF:5 * NBUF]
        gsems = rings[5 * NBUF:6 * NBUF]
        ssems = rings[6 * NBUF:7 * NBUF]

        c = lax.axis_index("c")
        s = lax.axis_index("s")
        wid = s * NC + c
        ebase = wid * ept
        _copy_rows(s, lambda d: zfeat_h.at[d], lambda d: acc_sh.at[d])
        if with_count:
            pltpu.sync_copy(zcnt_h, cnt_local.at[pl.ds(0, N)])
        plsc.subcore_barrier()

        ones = jnp.full((16,), 1.0, jnp.float32)

        def count_fn(dref):
            if with_count:
                for g in range(ch // 16):
                    plsc.addupdate_scatter(
                        cnt_local, [dref[pl.ds(g * 16, 16)]], ones)

        def start_idx(k, b):
            o = ebase + k * ch
            pltpu.make_async_copy(src_h.at[pl.ds(o, ch)], srcv[b],
                                  isems[b]).start()
            pltpu.make_async_copy(dst_h.at[pl.ds(o, ch)], dstv[b],
                                  isems[b]).start()

        def wait_idx(b):
            pltpu.make_async_copy(src_h.at[pl.ds(0, ch)], srcv[b],
                                  isems[b]).wait()
            pltpu.make_async_copy(dst_h.at[pl.ds(0, ch)], dstv[b],
                                  isems[b]).wait()

        def start_gather(b):
            wait_idx(b)
            pltpu.make_async_copy(table.at[srcv[b]], rows[b],
                                  gsems[b]).start()

        def wait_scatter(b):
            pltpu.make_async_copy(rows[b], acc_sh.at[dsc[b]],
                                  ssems[b]).wait()

        def drain(k, b):
            pltpu.make_async_copy(table.at[pl.ds(0, ch)], rows[b],
                                  gsems[b]).wait()
            for g in range(ch // 16):
                dsc[b][pl.ds(g * 16, 16)] = dstv[b][pl.ds(g * 16, 16)]
            pltpu.async_copy(rows[b], acc_sh.at[dsc[b]], ssems[b], add=True)
            count_fn(dsc[b])

        for b in range(NBUF):
            start_idx(b, b)
        for b in range(NBUF - 1):
            start_gather(b)

        @pl.loop(0, nmain, step=NBUF)
        def _(ci):
            for b in range(NBUF):
                k = ci + b
                drain(k, b)
                nk = k + NBUF

                @pl.when(nk < nchunk)
                def _():
                    start_idx(nk, b)

                m = k + NBUF - 1
                mb = (b + NBUF - 1) % NBUF

                @pl.when(m < nchunk)
                def _():
                    @pl.when(k > 0)
                    def _():
                        wait_scatter(mb)

                    start_gather(mb)

        for t in range(nmain, nchunk):
            drain(t, t % NBUF)
        for b in range(NBUF):
            wait_scatter(b)

        plsc.subcore_barrier()
        _copy_rows(s, lambda d: acc_sh.at[d], lambda d: acc_out.at[c, d])
        if with_count:
            pltpu.sync_copy(cnt_local.at[pl.ds(0, N)],
                            cnt_out.at[pl.ds(wid * N, N)])

    out_type = jax.ShapeDtypeStruct((NC, N, D), jnp.float32)
    if with_count:
        out_type = (out_type, jax.ShapeDtypeStruct((NW * N,), jnp.float32))
    scratch = (
        ([pltpu.VMEM((N + 16,), jnp.float32)] if with_count else [])
        + [pltpu.VMEM_SHARED((N + 16, D), jnp.float32)]
        + [pltpu.VMEM((ch,), jnp.int32)] * (3 * NBUF)
        + [pltpu.SemaphoreType.DMA] * NBUF
        + [pltpu.VMEM((ch, D), jnp.float32)] * NBUF
        + [pltpu.SemaphoreType.DMA] * (2 * NBUF)
    )
    cp = pltpu.CompilerParams()
    if "needs_layout_passes" in pltpu.CompilerParams.__dataclass_fields__:
        cp = dataclasses.replace(cp, needs_layout_passes=False)
    return pl.kernel(body, out_type=out_type, mesh=_mesh,
                     scratch_types=scratch, compiler_params=cp), ept


_sc_agg_count, _EPT1 = _make_sc_agg(80, with_count=True)
_sc_agg, _EPT2 = _make_sc_agg(80, with_count=False)


def _pad_edges(src, dst, ept):
    if ept == EPW:
        return src, dst
    pad = ept - EPW
    src_p = jnp.concatenate(
        [src.reshape(NW, EPW), jnp.zeros((NW, pad), jnp.int32)], axis=1)
    dst_p = jnp.concatenate(
        [dst.reshape(NW, EPW), jnp.full((NW, pad), NTRASH, jnp.int32)], axis=1)
    return src_p.reshape(-1), dst_p.reshape(-1)


_DN = (((1,), (1,)), ((), ()))


def _dense1_kernel(acc, cnt, x, w1l, b1l, w1r, out):
    a = acc[...]
    ssum = a[0] + a[1]
    deg = jnp.sum(cnt[...].reshape(NW, -1), axis=0)[:, None]
    mean = ssum / jnp.maximum(deg, 1.0)
    h = lax.dot_general(mean, w1l[...], _DN, preferred_element_type=jnp.float32)
    h = h + lax.dot_general(x[...], w1r[...], _DN,
                            preferred_element_type=jnp.float32)
    h = h + b1l[...]
    out[...] = jnp.maximum(h, 0.0)


def _dense2_kernel(acc, cnt, h1, w2l, b2l, w2r, wlin, blin, out):
    a = acc[...]
    ssum = a[0] + a[1]
    deg = jnp.sum(cnt[...].reshape(NW, -1), axis=0)[:, None]
    mean = ssum / jnp.maximum(deg, 1.0)
    h = lax.dot_general(mean, w2l[...], _DN, preferred_element_type=jnp.float32)
    h = h + lax.dot_general(h1[...], w2r[...], _DN,
                            preferred_element_type=jnp.float32)
    h = jnp.maximum(h + b2l[...], 0.0)
    out[...] = lax.dot_general(h, wlin[...], _DN,
                               preferred_element_type=jnp.float32) + blin[...]


_R = 1000

_dense1 = pl.pallas_call(
    _dense1_kernel,
    grid=(N // _R,),
    in_specs=[
        pl.BlockSpec((NC, _R, D), lambda i: (0, i, 0)),
        pl.BlockSpec((NW, 1, 1, _R), lambda i: (0, i, 0, 0)),
        pl.BlockSpec((_R, D), lambda i: (i, 0)),
        pl.BlockSpec((D, D), lambda i: (0, 0)),
        pl.BlockSpec((1, D), lambda i: (0, 0)),
        pl.BlockSpec((D, D), lambda i: (0, 0)),
    ],
    out_specs=pl.BlockSpec((_R, D), lambda i: (i, 0)),
    out_shape=jax.ShapeDtypeStruct((N, D), jnp.float32),
)

_dense2 = pl.pallas_call(
    _dense2_kernel,
    grid=(N // _R,),
    in_specs=[
        pl.BlockSpec((NC, _R, D), lambda i: (0, i, 0)),
        pl.BlockSpec((NW, 1, 1, _R), lambda i: (0, i, 0, 0)),
        pl.BlockSpec((_R, D), lambda i: (i, 0)),
        pl.BlockSpec((D, D), lambda i: (0, 0)),
        pl.BlockSpec((1, D), lambda i: (0, 0)),
        pl.BlockSpec((D, D), lambda i: (0, 0)),
        pl.BlockSpec((C_OUT, D), lambda i: (0, 0)),
        pl.BlockSpec((1, C_OUT), lambda i: (0, 0)),
    ],
    out_specs=pl.BlockSpec((_R, C_OUT), lambda i: (i, 0)),
    out_shape=jax.ShapeDtypeStruct((N, C_OUT), jnp.float32),
)


def kernel(x, edge_index, W1l, b1l, W1r, W2l, b2l, W2r, Wlin, blin):
    src = edge_index[0]
    dst = edge_index[1]
    src1, dst1 = _pad_edges(src, dst, _EPT1)
    src2, dst2 = _pad_edges(src, dst, _EPT2)
    zfeat = jnp.zeros((N, D), jnp.float32)
    zcnt = jnp.zeros((N,), jnp.float32)
    acc1, cnt = _sc_agg_count(x, src1, dst1, zfeat, zcnt)
    cnt = cnt.reshape(NW, N // _R, 1, _R)
    h1 = _dense1(acc1, cnt, x, W1l, b1l.reshape(1, D), W1r)
    acc2 = _sc_agg(h1, src2, dst2, zfeat)
    return _dense2(acc2, cnt, h1, W2l, b2l.reshape(1, D), W2r,
                   Wlin, blin.reshape(1, C_OUT))
